# baseline (speedup 1.0000x reference)
import functools

import jax
import jax.numpy as jnp
from jax import lax
from jax.experimental import pallas as pl
from jax.experimental.pallas import tpu as pltpu
from jax.experimental.pallas import tpu_sc as plsc

B, NC, NCAT, D = 16384, 13, 26, 64
NTOK = NC + NCAT
NROWS = NCAT * 100
EPS = 1e-5

_AGRID = 32
_AXH = B // 4
_AXB = _AXH // _AGRID
_STRIDE = 32


def _prep_body(xcat_ref, tbl_ref, gk_ref, bek_ref, fidx_ref, tbln_ref):
    i = pl.program_id(0)
    xc = xcat_ref[...]
    f = lax.broadcasted_iota(jnp.int32, (xc.shape[0], NCAT), 1)
    z = jnp.zeros((xc.shape[0], _STRIDE - NCAT), jnp.int32)
    pieces = []
    for k in range(4):
        sub = xc[:, k * NCAT:(k + 1) * NCAT]
        pieces += [jnp.clip(sub, 0, 99) + f * 100, z]
    fidx_ref[...] = jnp.concatenate(pieces, axis=1)

    @pl.when(i == 0)
    def _():
        t = tbl_ref[...]
        m = jnp.mean(t, axis=-1, keepdims=True)
        v = jnp.mean((t - m) ** 2, axis=-1, keepdims=True)
        tbln_ref[...] = (t - m) * lax.rsqrt(v + EPS) * gk_ref[...] + bek_ref[...]


def _prep(x_cat, tbl2d, g_cat, be_cat):
    vec = pl.BlockSpec((1, D), lambda i: (0, 0))
    return pl.pallas_call(
        _prep_body,
        grid=(_AGRID,),
        in_specs=[
            pl.BlockSpec((_AXB, 4 * NCAT), lambda i: (i, 0)),
            pl.BlockSpec((NROWS, D), lambda i: (0, 0)),
            vec, vec,
        ],
        out_specs=[
            pl.BlockSpec((_AXB, 128), lambda i: (i, 0)),
            pl.BlockSpec((NROWS, D), lambda i: (0, 0)),
        ],
        out_shape=[
            jax.ShapeDtypeStruct((_AXH, 128), jnp.int32),
            jax.ShapeDtypeStruct((NROWS, D), jnp.float32),
        ],
    )(x_cat, tbl2d, g_cat, be_cat)


_NCORES = 2
_NSUB = 16
_NWORK = _NCORES * _NSUB
_RPW = B // _NWORK
_NB = 16
_NCHUNK = _RPW // _NB

_sc_mesh = plsc.VectorSubcoreMesh(core_axis_name="c", subcore_axis_name="s")


_IPW = _RPW * _STRIDE


def _sc_body(fidx_hbm, tbl_hbm, out_hbm, idx_v, stg_v,
             sem_i, sem_g0, sem_g1, sem_w0, sem_w1):
    wid = lax.axis_index("s") * _NCORES + lax.axis_index("c")
    base = pl.multiple_of(wid * _RPW, _RPW)
    sem_g = (sem_g0, sem_g1)
    sem_w = (sem_w0, sem_w1)

    def gather_descs(g, slot):
        return [
            pltpu.make_async_copy(
                tbl_hbm.at[idx_v.at[pl.ds((g * _NB + i) * _STRIDE, NCAT)]],
                stg_v.at[slot, i],
                sem_g[slot],
            )
            for i in range(_NB)
        ]

    def out_desc(r0, slot):
        return pltpu.make_async_copy(
            stg_v.at[slot],
            out_hbm.at[pl.ds(r0, _NB), pl.ds(NC, NCAT)],
            sem_w[slot],
        )

    pltpu.sync_copy(
        fidx_hbm.at[pl.ds(pl.multiple_of(wid * _IPW, _IPW), _IPW)], idx_v
    )

    def chunk(g, carry):
        r0 = pl.multiple_of(base + g * _NB, _NB)

        def run(slot):
            @pl.when(g >= 2)
            def _():
                out_desc(r0, slot).wait()

            for h in gather_descs(g, slot):
                h.start()
            for h in gather_descs(g, slot):
                h.wait()
            out_desc(r0, slot).start()

        @pl.when(g % 2 == 0)
        def _():
            run(0)

        @pl.when(g % 2 == 1)
        def _():
            run(1)

        return carry

    lax.fori_loop(0, _NCHUNK, chunk, 0)
    out_desc(base, 0).wait()
    out_desc(base, 1).wait()


_sc_gather = functools.partial(
    pl.kernel,
    out_type=jax.ShapeDtypeStruct((B, NTOK, D), jnp.float32),
    mesh=_sc_mesh,
    compiler_params=pltpu.CompilerParams(use_tc_tiling_on_sc=False),
    scratch_types=[
        pltpu.VMEM((_RPW * _STRIDE,), jnp.int32),
        pltpu.VMEM((2, _NB, NCAT, D), jnp.float32),
        pltpu.SemaphoreType.DMA,
        pltpu.SemaphoreType.DMA,
        pltpu.SemaphoreType.DMA,
        pltpu.SemaphoreType.DMA,
        pltpu.SemaphoreType.DMA,
    ],
)(_sc_body)


_CGRID = 32
_CB = B // _CGRID


def _cont_body(xc_ref, cat_ref, w1_ref, b1_ref, g_ref, be_ref, out_ref,
               stg_ref, sem):
    del cat_ref
    i = pl.program_id(0)
    w1 = w1_ref[...]
    b1 = b1_ref[...]
    g = g_ref[...]
    be = be_ref[...]
    mw = jnp.mean(w1)
    mb = jnp.mean(b1)
    wc = w1 - mw
    bc = b1 - mb
    p2 = jnp.mean(wc * wc)
    pq = jnp.mean(wc * bc)
    q2 = jnp.mean(bc * bc)
    x = jnp.clip(xc_ref[...], -10.0, 10.0)
    s = lax.rsqrt(x * x * p2 + 2.0 * x * pq + q2 + EPS)
    a = s * x
    P = (wc * g)[0]
    Q = (bc * g)[0]
    R = be[0]
    stg_ref[...] = (
        a[:, :, None] * P[None, None, :]
        + s[:, :, None] * Q[None, None, :]
        + R[None, None, :]
    )
    cp = pltpu.make_async_copy(
        stg_ref, out_ref.at[pl.ds(i * _CB, _CB), pl.ds(0, NC)], sem
    )
    cp.start()
    cp.wait()


def _cont(x_cont, full, w1, b1, g_cont, be_cont):
    vec = pl.BlockSpec((1, D), lambda i: (0, 0))
    return pl.pallas_call(
        _cont_body,
        grid=(_CGRID,),
        in_specs=[
            pl.BlockSpec((_CB, NC), lambda i: (i, 0)),
            pl.BlockSpec(memory_space=pl.ANY),
            vec, vec, vec, vec,
        ],
        out_specs=pl.BlockSpec(memory_space=pl.ANY),
        out_shape=jax.ShapeDtypeStruct((B, NTOK, D), jnp.float32),
        input_output_aliases={1: 0},
        scratch_shapes=[
            pltpu.VMEM((_CB, NC, D), jnp.float32),
            pltpu.SemaphoreType.DMA,
        ],
    )(x_cont, full, w1, b1, g_cont, be_cont)


def kernel(x_cont, x_cat, w1, b1, g_cont, be_cont, tables, g_cat, be_cat):
    tbl2d = tables.reshape(NROWS, D)
    xcat2d = x_cat.reshape(_AXH, 4 * NCAT)
    fidx2d, tbln = _prep(xcat2d, tbl2d, g_cat.reshape(1, D), be_cat.reshape(1, D))
    full = _sc_gather(fidx2d.reshape(B * _STRIDE), tbln)
    return _cont(
        x_cont, full,
        w1.reshape(1, D), b1.reshape(1, D),
        g_cont.reshape(1, D), be_cont.reshape(1, D),
    )


# speedup vs baseline: 1.0897x; 1.0897x over previous
import functools

import jax
import jax.numpy as jnp
from jax import lax
from jax.experimental import pallas as pl
from jax.experimental.pallas import tpu as pltpu
from jax.experimental.pallas import tpu_sc as plsc

B, NC, NCAT, D = 16384, 13, 26, 64
NTOK = NC + NCAT
NROWS = NCAT * 100
EPS = 1e-5

_AGRID = 32
_AXH = B // 4
_AXB = _AXH // _AGRID
_STRIDE = 32


def _prep_body(xcat_ref, tbl_ref, gk_ref, bek_ref, fidx_ref, tbln_ref):
    i = pl.program_id(0)
    xc = xcat_ref[...]
    f = lax.broadcasted_iota(jnp.int32, (xc.shape[0], NCAT), 1)
    z = jnp.zeros((xc.shape[0], _STRIDE - NCAT), jnp.int32)
    pieces = []
    for k in range(4):
        sub = xc[:, k * NCAT:(k + 1) * NCAT]
        pieces += [jnp.clip(sub, 0, 99) + f * 100, z]
    fidx_ref[...] = jnp.concatenate(pieces, axis=1)

    @pl.when(i == 0)
    def _():
        t = tbl_ref[...]
        m = jnp.mean(t, axis=-1, keepdims=True)
        v = jnp.mean((t - m) ** 2, axis=-1, keepdims=True)
        tbln_ref[...] = (t - m) * lax.rsqrt(v + EPS) * gk_ref[...] + bek_ref[...]


def _prep(x_cat, tbl2d, g_cat, be_cat):
    vec = pl.BlockSpec((1, D), lambda i: (0, 0))
    return pl.pallas_call(
        _prep_body,
        grid=(_AGRID,),
        in_specs=[
            pl.BlockSpec((_AXB, 4 * NCAT), lambda i: (i, 0)),
            pl.BlockSpec((NROWS, D), lambda i: (0, 0)),
            vec, vec,
        ],
        out_specs=[
            pl.BlockSpec((_AXB, 128), lambda i: (i, 0)),
            pl.BlockSpec((NROWS, D), lambda i: (0, 0)),
        ],
        out_shape=[
            jax.ShapeDtypeStruct((_AXH, 128), jnp.int32),
            jax.ShapeDtypeStruct((NROWS, D), jnp.float32),
        ],
    )(x_cat, tbl2d, g_cat, be_cat)


_NCORES = 2
_NSUB = 16
_NWORK = _NCORES * _NSUB
_RPW = B // _NWORK
_NB = 16
_NCHUNK = _RPW // _NB
_TROWS = 200

_sc_mesh = plsc.VectorSubcoreMesh(core_axis_name="c", subcore_axis_name="s")


_IPW = _RPW * _STRIDE


def _sc_body(fidx_hbm, tbl_hbm, cat_hbm, idx_v, stg_v, shr_tbl,
             sem_i, sem_g0, sem_g1, sem_w0, sem_w1):
    sid = lax.axis_index("s")
    wid = sid * _NCORES + lax.axis_index("c")
    base = pl.multiple_of(wid * _RPW, _RPW)
    sem_g = (sem_g0, sem_g1)
    sem_w = (sem_w0, sem_w1)

    @pl.when(sid < 13)
    def _():
        r = pl.ds(sid * _TROWS, _TROWS)
        pltpu.sync_copy(tbl_hbm.at[r], shr_tbl.at[r])

    plsc.subcore_barrier()

    def gather_descs(g, slot):
        return [
            pltpu.make_async_copy(
                shr_tbl.at[idx_v.at[pl.ds((g * _NB + i) * _STRIDE, NCAT)]],
                stg_v.at[slot, i],
                sem_g[slot],
            )
            for i in range(_NB)
        ]

    def out_desc(r0, slot):
        return pltpu.make_async_copy(
            stg_v.at[slot], cat_hbm.at[pl.ds(r0, _NB)], sem_w[slot]
        )

    pltpu.sync_copy(
        fidx_hbm.at[pl.ds(pl.multiple_of(wid * _IPW, _IPW), _IPW)], idx_v
    )

    def chunk(g, carry):
        r0 = pl.multiple_of(base + g * _NB, _NB)

        def run(slot):
            @pl.when(g >= 2)
            def _():
                out_desc(r0, slot).wait()

            for h in gather_descs(g, slot):
                h.start()
            for h in gather_descs(g, slot):
                h.wait()
            out_desc(r0, slot).start()

        @pl.when(g % 2 == 0)
        def _():
            run(0)

        @pl.when(g % 2 == 1)
        def _():
            run(1)

        return carry

    lax.fori_loop(0, _NCHUNK, chunk, 0)
    out_desc(base, 0).wait()
    out_desc(base, 1).wait()


_sc_gather = functools.partial(
    pl.kernel,
    out_type=jax.ShapeDtypeStruct((B, NCAT, D), jnp.float32),
    mesh=_sc_mesh,
    compiler_params=pltpu.CompilerParams(use_tc_tiling_on_sc=False),
    scratch_types=[
        pltpu.VMEM((_RPW * _STRIDE,), jnp.int32),
        pltpu.VMEM((2, _NB, NCAT, D), jnp.float32),
        pltpu.VMEM_SHARED((NROWS, D), jnp.float32),
        pltpu.SemaphoreType.DMA,
        pltpu.SemaphoreType.DMA,
        pltpu.SemaphoreType.DMA,
        pltpu.SemaphoreType.DMA,
        pltpu.SemaphoreType.DMA,
    ],
)(_sc_body)


_CGRID = 32
_CB = B // _CGRID


def _cont_body(xc_ref, cat_ref, w1_ref, b1_ref, g_ref, be_ref, out_ref):
    w1 = w1_ref[...]
    b1 = b1_ref[...]
    g = g_ref[...]
    be = be_ref[...]
    mw = jnp.mean(w1)
    mb = jnp.mean(b1)
    wc = w1 - mw
    bc = b1 - mb
    p2 = jnp.mean(wc * wc)
    pq = jnp.mean(wc * bc)
    q2 = jnp.mean(bc * bc)
    x = jnp.clip(xc_ref[...], -10.0, 10.0)
    s = lax.rsqrt(x * x * p2 + 2.0 * x * pq + q2 + EPS)
    a = s * x
    P = (wc * g)[0]
    Q = (bc * g)[0]
    R = be[0]
    cont = (
        a[:, :, None] * P[None, None, :]
        + s[:, :, None] * Q[None, None, :]
        + R[None, None, :]
    )
    out_ref[...] = jnp.concatenate([cont, cat_ref[...]], axis=1)


def _cont(x_cont, cat64, w1, b1, g_cont, be_cont):
    vec = pl.BlockSpec((1, D), lambda i: (0, 0))
    return pl.pallas_call(
        _cont_body,
        grid=(_CGRID,),
        in_specs=[
            pl.BlockSpec((_CB, NC), lambda i: (i, 0)),
            pl.BlockSpec((_CB, NCAT, D), lambda i: (i, 0, 0)),
            vec, vec, vec, vec,
        ],
        out_specs=pl.BlockSpec((_CB, NTOK, D), lambda i: (i, 0, 0)),
        out_shape=jax.ShapeDtypeStruct((B, NTOK, D), jnp.float32),
    )(x_cont, cat64, w1, b1, g_cont, be_cont)


def kernel(x_cont, x_cat, w1, b1, g_cont, be_cont, tables, g_cat, be_cat):
    tbl2d = tables.reshape(NROWS, D)
    xcat2d = x_cat.reshape(_AXH, 4 * NCAT)
    fidx2d, tbln = _prep(xcat2d, tbl2d, g_cat.reshape(1, D), be_cat.reshape(1, D))
    cat64 = _sc_gather(fidx2d.reshape(B * _STRIDE), tbln)
    return _cont(
        x_cont, cat64,
        w1.reshape(1, D), b1.reshape(1, D),
        g_cont.reshape(1, D), be_cont.reshape(1, D),
    )


# speedup vs baseline: 1.1798x; 1.0827x over previous
import functools

import jax
import jax.numpy as jnp
from jax import lax
from jax.experimental import pallas as pl
from jax.experimental.pallas import tpu as pltpu
from jax.experimental.pallas import tpu_sc as plsc

B, NC, NCAT, D = 16384, 13, 26, 64
HB = B // 2
NTOK = NC + NCAT
NROWS = NCAT * 100
WROW = 2 * D
EPS = 1e-5

_AGRID = 32
_AXH = B // 4
_AXB = _AXH // _AGRID
_STRIDE = 32


def _prep_body(xcat_ref, tbl_ref, gk_ref, bek_ref, fidx_ref, tbln_ref):
    i = pl.program_id(0)
    xc = xcat_ref[...]
    f = lax.broadcasted_iota(jnp.int32, (xc.shape[0], NCAT), 1)
    z = jnp.zeros((xc.shape[0], _STRIDE - NCAT), jnp.int32)
    pieces = []
    for k in range(4):
        sub = xc[:, k * NCAT:(k + 1) * NCAT]
        pieces += [jnp.clip(sub, 0, 99) + f * 100, z]
    fidx_ref[...] = jnp.concatenate(pieces, axis=1)

    @pl.when(i == 0)
    def _():
        t = tbl_ref[...]
        m = jnp.mean(t, axis=-1, keepdims=True)
        v = jnp.mean((t - m) ** 2, axis=-1, keepdims=True)
        n = (t - m) * lax.rsqrt(v + EPS) * gk_ref[...] + bek_ref[...]
        tbln_ref[...] = jnp.concatenate([n, jnp.zeros_like(n)], axis=-1)


def _prep(x_cat, tbl2d, g_cat, be_cat):
    vec = pl.BlockSpec((1, D), lambda i: (0, 0))
    return pl.pallas_call(
        _prep_body,
        grid=(_AGRID,),
        in_specs=[
            pl.BlockSpec((_AXB, 4 * NCAT), lambda i: (i, 0)),
            pl.BlockSpec((NROWS, D), lambda i: (0, 0)),
            vec, vec,
        ],
        out_specs=[
            pl.BlockSpec((_AXB, 128), lambda i: (i, 0)),
            pl.BlockSpec((NROWS, WROW), lambda i: (0, 0)),
        ],
        out_shape=[
            jax.ShapeDtypeStruct((_AXH, 128), jnp.int32),
            jax.ShapeDtypeStruct((NROWS, WROW), jnp.float32),
        ],
    )(x_cat, tbl2d, g_cat, be_cat)


_NCORES = 2
_NSUB = 16
_NWORK = _NCORES * _NSUB
_RPW = HB // _NWORK
_NB = 8
_NCHUNK = _RPW // _NB
_IPW = _RPW * _STRIDE

_sc_mesh = plsc.VectorSubcoreMesh(core_axis_name="c", subcore_axis_name="s")


def _sc_body(hoff, fidx_hbm, tbl_hbm, cat_hbm, idx_v, stg_v,
             sem_i, sem_g0, sem_g1, sem_w0, sem_w1):
    wid = lax.axis_index("s") * _NCORES + lax.axis_index("c")
    base = pl.multiple_of(wid * _RPW, _RPW)
    sem_g = (sem_g0, sem_g1)
    sem_w = (sem_w0, sem_w1)

    def gather_descs(g, slot):
        return [
            pltpu.make_async_copy(
                tbl_hbm.at[idx_v.at[pl.ds((g * _NB + i) * _STRIDE, NCAT)]],
                stg_v.at[slot, i],
                sem_g[slot],
            )
            for i in range(_NB)
        ]

    def out_desc(r0, slot):
        return pltpu.make_async_copy(
            stg_v.at[slot], cat_hbm.at[pl.ds(r0, _NB)], sem_w[slot]
        )

    ioff = hoff * HB * _STRIDE + wid * _IPW
    pltpu.sync_copy(
        fidx_hbm.at[pl.ds(pl.multiple_of(ioff, _IPW), _IPW)], idx_v
    )

    def chunk(g, carry):
        r0 = pl.multiple_of(base + g * _NB, _NB)

        def run(slot):
            @pl.when(g >= 2)
            def _():
                out_desc(r0, slot).wait()

            for h in gather_descs(g, slot):
                h.start()
            for h in gather_descs(g, slot):
                h.wait()
            out_desc(r0, slot).start()

        @pl.when(g % 2 == 0)
        def _():
            run(0)

        @pl.when(g % 2 == 1)
        def _():
            run(1)

        return carry

    lax.fori_loop(0, _NCHUNK, chunk, 0)
    out_desc(base, 0).wait()
    out_desc(base, 1).wait()


def _make_sc_gather(hoff):
    return functools.partial(
        pl.kernel,
        out_type=jax.ShapeDtypeStruct((HB, NCAT, WROW), jnp.float32),
        mesh=_sc_mesh,
        scratch_types=[
            pltpu.VMEM((_IPW,), jnp.int32),
            pltpu.VMEM((2, _NB, NCAT, WROW), jnp.float32),
            pltpu.SemaphoreType.DMA,
            pltpu.SemaphoreType.DMA,
            pltpu.SemaphoreType.DMA,
            pltpu.SemaphoreType.DMA,
            pltpu.SemaphoreType.DMA,
        ],
    )(functools.partial(_sc_body, hoff))


_sc_gather_h0 = _make_sc_gather(0)
_sc_gather_h1 = _make_sc_gather(1)


_CGRID = 16
_CB = HB // _CGRID


def _cont_math(xc_ref, w1_ref, b1_ref, g_ref, be_ref):
    w1 = w1_ref[...]
    b1 = b1_ref[...]
    g = g_ref[...]
    be = be_ref[...]
    mw = jnp.mean(w1)
    mb = jnp.mean(b1)
    wc = w1 - mw
    bc = b1 - mb
    p2 = jnp.mean(wc * wc)
    pq = jnp.mean(wc * bc)
    q2 = jnp.mean(bc * bc)
    x = jnp.clip(xc_ref[...], -10.0, 10.0)
    s = lax.rsqrt(x * x * p2 + 2.0 * x * pq + q2 + EPS)
    a = s * x
    P = (wc * g)[0]
    Q = (bc * g)[0]
    R = be[0]
    return (
        a[:, :, None] * P[None, None, :]
        + s[:, :, None] * Q[None, None, :]
        + R[None, None, :]
    )


def _cont_h0_body(xc_ref, cat_ref, w1_ref, b1_ref, g_ref, be_ref, out_ref):
    cont = _cont_math(xc_ref, w1_ref, b1_ref, g_ref, be_ref)
    out_ref[...] = jnp.concatenate([cont, cat_ref[:, :, 0:D]], axis=1)


def _cont_h1_body(xc_ref, cat_ref, w1_ref, b1_ref, g_ref, be_ref, prev_ref,
                  out_ref):
    del prev_ref
    cont = _cont_math(xc_ref, w1_ref, b1_ref, g_ref, be_ref)
    out_ref[...] = jnp.concatenate([cont, cat_ref[:, :, 0:D]], axis=1)


def _cont_h0(x_cont, cat128, w1, b1, g_cont, be_cont):
    vec = pl.BlockSpec((1, D), lambda i: (0, 0))
    return pl.pallas_call(
        _cont_h0_body,
        grid=(_CGRID,),
        in_specs=[
            pl.BlockSpec((_CB, NC), lambda i: (i, 0)),
            pl.BlockSpec((_CB, NCAT, WROW), lambda i: (i, 0, 0)),
            vec, vec, vec, vec,
        ],
        out_specs=pl.BlockSpec((_CB, NTOK, D), lambda i: (i, 0, 0)),
        out_shape=jax.ShapeDtypeStruct((B, NTOK, D), jnp.float32),
    )(x_cont, cat128, w1, b1, g_cont, be_cont)


def _cont_h1(x_cont, cat128, w1, b1, g_cont, be_cont, prev):
    vec = pl.BlockSpec((1, D), lambda i: (0, 0))
    nh = _CGRID
    return pl.pallas_call(
        _cont_h1_body,
        grid=(_CGRID,),
        in_specs=[
            pl.BlockSpec((_CB, NC), lambda i: (i + nh, 0)),
            pl.BlockSpec((_CB, NCAT, WROW), lambda i: (i, 0, 0)),
            vec, vec, vec, vec,
            pl.BlockSpec(memory_space=pl.ANY),
        ],
        out_specs=pl.BlockSpec((_CB, NTOK, D), lambda i: (i + nh, 0, 0)),
        out_shape=jax.ShapeDtypeStruct((B, NTOK, D), jnp.float32),
        input_output_aliases={6: 0},
    )(x_cont, cat128, w1, b1, g_cont, be_cont, prev)


def kernel(x_cont, x_cat, w1, b1, g_cont, be_cont, tables, g_cat, be_cat):
    tbl2d = tables.reshape(NROWS, D)
    xcat2d = x_cat.reshape(_AXH, 4 * NCAT)
    fidx2d, tbln = _prep(xcat2d, tbl2d, g_cat.reshape(1, D), be_cat.reshape(1, D))
    fidx = fidx2d.reshape(B * _STRIDE)
    w1r, b1r = w1.reshape(1, D), b1.reshape(1, D)
    gr, ber = g_cont.reshape(1, D), be_cont.reshape(1, D)
    cat_h0 = _sc_gather_h0(fidx, tbln)
    cat_h1 = _sc_gather_h1(fidx, tbln)
    out = _cont_h0(x_cont, cat_h0, w1r, b1r, gr, ber)
    return _cont_h1(x_cont, cat_h1, w1r, b1r, gr, ber, out)


# speedup vs baseline: 1.3903x; 1.1784x over previous
import functools

import jax
import jax.numpy as jnp
from jax import lax
from jax.experimental import pallas as pl
from jax.experimental.pallas import tpu as pltpu
from jax.experimental.pallas import tpu_sc as plsc

B, NC, NCAT, D = 16384, 13, 26, 64
HB = B // 2
NTOK = NC + NCAT
NROWS = NCAT * 100
WROW = 2 * D
EPS = 1e-5

_AGRID = 32
_AXH = B // 4
_AXB = _AXH // _AGRID
_STRIDE = 32


def _prep_body(xcat_ref, tbl_ref, gk_ref, bek_ref, fidx_ref, tbln_ref):
    i = pl.program_id(0)
    xc = xcat_ref[...]
    f = lax.broadcasted_iota(jnp.int32, (xc.shape[0], NCAT), 1)
    z = jnp.zeros((xc.shape[0], _STRIDE - NCAT), jnp.int32)
    pieces = []
    for k in range(4):
        sub = xc[:, k * NCAT:(k + 1) * NCAT]
        pieces += [jnp.clip(sub, 0, 99) + f * 100, z]
    fidx_ref[...] = jnp.concatenate(pieces, axis=1)

    @pl.when(i == 0)
    def _():
        t = tbl_ref[...]
        m = jnp.mean(t, axis=-1, keepdims=True)
        v = jnp.mean((t - m) ** 2, axis=-1, keepdims=True)
        n = (t - m) * lax.rsqrt(v + EPS) * gk_ref[...] + bek_ref[...]
        tbln_ref[...] = jnp.concatenate([n, jnp.zeros_like(n)], axis=-1)


def _prep(x_cat, tbl2d, g_cat, be_cat):
    vec = pl.BlockSpec((1, D), lambda i: (0, 0))
    return pl.pallas_call(
        _prep_body,
        grid=(_AGRID,),
        in_specs=[
            pl.BlockSpec((_AXB, 4 * NCAT), lambda i: (i, 0)),
            pl.BlockSpec((NROWS, D), lambda i: (0, 0)),
            vec, vec,
        ],
        out_specs=[
            pl.BlockSpec((_AXB, 128), lambda i: (i, 0)),
            pl.BlockSpec((NROWS, WROW), lambda i: (0, 0)),
        ],
        out_shape=[
            jax.ShapeDtypeStruct((_AXH, 128), jnp.int32),
            jax.ShapeDtypeStruct((NROWS, WROW), jnp.float32),
        ],
    )(x_cat, tbl2d, g_cat, be_cat)


_NCORES = 2
_NSUB = 16
_NWORK = _NCORES * _NSUB
_RPW = HB // _NWORK
_NB = 8
_NCHUNK = _RPW // _NB
_IPW = _RPW * _STRIDE

_sc_mesh = plsc.VectorSubcoreMesh(core_axis_name="c", subcore_axis_name="s")


def _sc_body(hoff, fidx_hbm, tbl_hbm, cat_hbm, idx_v, stg_v, shr_tbl,
             sem_i, sem_g0, sem_g1, sem_w0, sem_w1):
    sid = lax.axis_index("s")
    wid = sid * _NCORES + lax.axis_index("c")
    base = pl.multiple_of(wid * _RPW, _RPW)
    sem_g = (sem_g0, sem_g1)
    sem_w = (sem_w0, sem_w1)

    @pl.when(sid < 13)
    def _():
        r = pl.ds(sid * 200, 200)
        pltpu.sync_copy(tbl_hbm.at[r], shr_tbl.at[r])

    plsc.subcore_barrier()

    def gather_descs(g, slot):
        return [
            pltpu.make_async_copy(
                shr_tbl.at[idx_v.at[pl.ds((g * _NB + i) * _STRIDE, NCAT)]],
                stg_v.at[slot, i],
                sem_g[slot],
            )
            for i in range(_NB)
        ]

    def out_desc(r0, slot):
        return pltpu.make_async_copy(
            stg_v.at[slot], cat_hbm.at[pl.ds(r0, _NB)], sem_w[slot]
        )

    ioff = hoff * HB * _STRIDE + wid * _IPW
    pltpu.sync_copy(
        fidx_hbm.at[pl.ds(pl.multiple_of(ioff, _IPW), _IPW)], idx_v
    )

    def chunk(g, carry):
        r0 = pl.multiple_of(base + g * _NB, _NB)

        def run(slot):
            @pl.when(g >= 2)
            def _():
                out_desc(r0, slot).wait()

            for h in gather_descs(g, slot):
                h.start()
            for h in gather_descs(g, slot):
                h.wait()
            out_desc(r0, slot).start()

        @pl.when(g % 2 == 0)
        def _():
            run(0)

        @pl.when(g % 2 == 1)
        def _():
            run(1)

        return carry

    lax.fori_loop(0, _NCHUNK, chunk, 0)
    out_desc(base, 0).wait()
    out_desc(base, 1).wait()


def _make_sc_gather(hoff):
    return functools.partial(
        pl.kernel,
        out_type=jax.ShapeDtypeStruct((HB, NCAT, WROW), jnp.float32),
        mesh=_sc_mesh,
        scratch_types=[
            pltpu.VMEM((_IPW,), jnp.int32),
            pltpu.VMEM((2, _NB, NCAT, WROW), jnp.float32),
            pltpu.VMEM_SHARED((NROWS, WROW), jnp.float32),
            pltpu.SemaphoreType.DMA,
            pltpu.SemaphoreType.DMA,
            pltpu.SemaphoreType.DMA,
            pltpu.SemaphoreType.DMA,
            pltpu.SemaphoreType.DMA,
        ],
    )(functools.partial(_sc_body, hoff))


_sc_gather_h0 = _make_sc_gather(0)
_sc_gather_h1 = _make_sc_gather(1)


_CGRID = 16
_CB = HB // _CGRID


def _cont_math(xc_ref, w1_ref, b1_ref, g_ref, be_ref):
    w1 = w1_ref[...]
    b1 = b1_ref[...]
    g = g_ref[...]
    be = be_ref[...]
    mw = jnp.mean(w1)
    mb = jnp.mean(b1)
    wc = w1 - mw
    bc = b1 - mb
    p2 = jnp.mean(wc * wc)
    pq = jnp.mean(wc * bc)
    q2 = jnp.mean(bc * bc)
    x = jnp.clip(xc_ref[...], -10.0, 10.0)
    s = lax.rsqrt(x * x * p2 + 2.0 * x * pq + q2 + EPS)
    a = s * x
    P = (wc * g)[0]
    Q = (bc * g)[0]
    R = be[0]
    return (
        a[:, :, None] * P[None, None, :]
        + s[:, :, None] * Q[None, None, :]
        + R[None, None, :]
    )


def _cont_h0_body(xc_ref, cat_ref, w1_ref, b1_ref, g_ref, be_ref, out_ref):
    cont = _cont_math(xc_ref, w1_ref, b1_ref, g_ref, be_ref)
    out_ref[...] = jnp.concatenate([cont, cat_ref[:, :, 0:D]], axis=1)


def _cont_h1_body(xc_ref, cat_ref, w1_ref, b1_ref, g_ref, be_ref, prev_ref,
                  out_ref):
    del prev_ref
    cont = _cont_math(xc_ref, w1_ref, b1_ref, g_ref, be_ref)
    out_ref[...] = jnp.concatenate([cont, cat_ref[:, :, 0:D]], axis=1)


def _cont_h0(x_cont, cat128, w1, b1, g_cont, be_cont):
    vec = pl.BlockSpec((1, D), lambda i: (0, 0))
    return pl.pallas_call(
        _cont_h0_body,
        grid=(_CGRID,),
        in_specs=[
            pl.BlockSpec((_CB, NC), lambda i: (i, 0)),
            pl.BlockSpec((_CB, NCAT, WROW), lambda i: (i, 0, 0)),
            vec, vec, vec, vec,
        ],
        out_specs=pl.BlockSpec((_CB, NTOK, D), lambda i: (i, 0, 0)),
        out_shape=jax.ShapeDtypeStruct((B, NTOK, D), jnp.float32),
    )(x_cont, cat128, w1, b1, g_cont, be_cont)


def _cont_h1(x_cont, cat128, w1, b1, g_cont, be_cont, prev):
    vec = pl.BlockSpec((1, D), lambda i: (0, 0))
    nh = _CGRID
    return pl.pallas_call(
        _cont_h1_body,
        grid=(_CGRID,),
        in_specs=[
            pl.BlockSpec((_CB, NC), lambda i: (i + nh, 0)),
            pl.BlockSpec((_CB, NCAT, WROW), lambda i: (i, 0, 0)),
            vec, vec, vec, vec,
            pl.BlockSpec(memory_space=pl.ANY),
        ],
        out_specs=pl.BlockSpec((_CB, NTOK, D), lambda i: (i + nh, 0, 0)),
        out_shape=jax.ShapeDtypeStruct((B, NTOK, D), jnp.float32),
        input_output_aliases={6: 0},
    )(x_cont, cat128, w1, b1, g_cont, be_cont, prev)


def kernel(x_cont, x_cat, w1, b1, g_cont, be_cont, tables, g_cat, be_cat):
    tbl2d = tables.reshape(NROWS, D)
    xcat2d = x_cat.reshape(_AXH, 4 * NCAT)
    fidx2d, tbln = _prep(xcat2d, tbl2d, g_cat.reshape(1, D), be_cat.reshape(1, D))
    fidx = fidx2d.reshape(B * _STRIDE)
    w1r, b1r = w1.reshape(1, D), b1.reshape(1, D)
    gr, ber = g_cont.reshape(1, D), be_cont.reshape(1, D)
    cat_h0 = _sc_gather_h0(fidx, tbln)
    cat_h1 = _sc_gather_h1(fidx, tbln)
    out = _cont_h0(x_cont, cat_h0, w1r, b1r, gr, ber)
    return _cont_h1(x_cont, cat_h1, w1r, b1r, gr, ber, out)
